# baseline (speedup 1.0000x reference)
import jax
import jax.numpy as jnp
from jax import lax
from jax.experimental import pallas as pl
from jax.experimental.pallas import tpu as pltpu
from jax.experimental.pallas import tpu_sc as plsc

B = 16384
F = 26
FIELD = 40000
TOTAL_VOCAB = F * FIELD
NC = 2
NS = 16
L = 16
NW = NC * NS
BPW = B // NW
PER_W = F * BPW
NCHUNK = BPW // L
VSLICE = 65024
VLAST = TOTAL_VOCAB - (NS - 1) * VSLICE


def _sc_body(x_hbm, w_hbm, table_hbm, bias_hbm, out_hbm,
             idx_v, w_v, val_v, out_v, bias_v, spmem_t, sem, sem2, semw):
    c = lax.axis_index("c")
    s = lax.axis_index("s")
    wid = s * NC + c
    base = wid * BPW

    @pl.when(s < NS - 1)
    def _stage_main():
        pltpu.make_async_copy(
            table_hbm.at[0, pl.ds(s * VSLICE, VSLICE)],
            spmem_t.at[0, pl.ds(s * VSLICE, VSLICE)], sem2).start()

    @pl.when(s == NS - 1)
    def _stage_last():
        pltpu.make_async_copy(
            table_hbm.at[0, pl.ds((NS - 1) * VSLICE, VLAST)],
            spmem_t.at[0, pl.ds((NS - 1) * VSLICE, VLAST)], sem2).start()

    xcopies, wcopies = [], []
    for f in range(F):
        xcopies.append(pltpu.make_async_copy(
            x_hbm.at[f, pl.ds(base, BPW)], idx_v.at[pl.ds(f * BPW, BPW)], sem))
        wcopies.append(pltpu.make_async_copy(
            w_hbm.at[0, pl.ds(f * B + base, BPW)], w_v.at[pl.ds(f * BPW, BPW)],
            semw))
    for cp in xcopies:
        cp.start()
    for cp in wcopies:
        cp.start()
    pltpu.sync_copy(bias_hbm, bias_v)
    for cp in xcopies:
        cp.wait()

    @pl.when(s < NS - 1)
    def _wait_main():
        pltpu.make_async_copy(
            table_hbm.at[0, pl.ds(s * VSLICE, VSLICE)],
            spmem_t.at[0, pl.ds(s * VSLICE, VSLICE)], sem2).wait()

    @pl.when(s == NS - 1)
    def _wait_last():
        pltpu.make_async_copy(
            table_hbm.at[0, pl.ds((NS - 1) * VSLICE, VLAST)],
            spmem_t.at[0, pl.ds((NS - 1) * VSLICE, VLAST)], sem2).wait()

    FH = F // 2

    def _offsets(flo, fhi):
        def _addf(f, _):
            off = f * jnp.int32(FIELD)
            fb = f * BPW

            def _add(i, _, off=off, fb=fb):
                sl = pl.ds(fb + i * L, L)
                idx_v[sl] = idx_v[sl] + off
                return _

            return lax.fori_loop(0, NCHUNK, _add, _, unroll=4)

        lax.fori_loop(max(flo, 1), fhi, _addf, 0)

    _offsets(0, FH)

    plsc.subcore_barrier()
    g1 = pltpu.make_async_copy(
        spmem_t.at[0].at[idx_v.at[pl.ds(0, FH * BPW)]],
        val_v.at[pl.ds(0, FH * BPW)], sem)
    g1.start()
    _offsets(FH, F)
    g2 = pltpu.make_async_copy(
        spmem_t.at[0].at[idx_v.at[pl.ds(FH * BPW, (F - FH) * BPW)]],
        val_v.at[pl.ds(FH * BPW, (F - FH) * BPW)], sem)
    g2.start()
    for cp in wcopies:
        cp.wait()
    g1.wait()

    def _reduce_a(i, _):
        sl0 = pl.ds(i * L, L)
        acc = bias_v[...] + val_v[sl0] * w_v[sl0]
        for f in range(1, FH):
            sl = pl.ds(f * BPW + i * L, L)
            acc = acc + val_v[sl] * w_v[sl]
        out_v[sl0] = acc
        return _

    lax.fori_loop(0, NCHUNK, _reduce_a, 0)
    g2.wait()

    def _reduce_b(i, _):
        sl0 = pl.ds(i * L, L)
        acc = out_v[sl0]
        for f in range(FH, F):
            sl = pl.ds(f * BPW + i * L, L)
            acc = acc + val_v[sl] * w_v[sl]
        out_v[sl0] = acc
        return _

    lax.fori_loop(0, NCHUNK, _reduce_b, 0)

    pltpu.sync_copy(out_v, out_hbm.at[pl.ds(base, BPW)])


@jax.jit
def kernel(x, weight, fc_table, bias):
    x_t = x.astype(jnp.int32).T
    w_t = lax.reshape(weight, (1, B * F), dimensions=(2, 1, 0))
    table2 = fc_table.reshape(1, TOTAL_VOCAB)
    bias16 = jnp.broadcast_to(bias.reshape(1), (L,))

    mesh = plsc.VectorSubcoreMesh(core_axis_name="c", subcore_axis_name="s")
    out = pl.kernel(
        _sc_body,
        mesh=mesh,
        out_type=jax.ShapeDtypeStruct((B,), jnp.float32),
        scratch_types=[
            pltpu.VMEM((PER_W,), jnp.int32),
            pltpu.VMEM((PER_W,), jnp.float32),
            pltpu.VMEM((PER_W,), jnp.float32),
            pltpu.VMEM((BPW,), jnp.float32),
            pltpu.VMEM((L,), jnp.float32),
            pltpu.VMEM_SHARED((1, TOTAL_VOCAB), jnp.float32),
            pltpu.SemaphoreType.DMA,
            pltpu.SemaphoreType.DMA,
            pltpu.SemaphoreType.DMA,
        ],
    )(x_t, w_t, table2, bias16)
    return out.reshape(B, 1)


# speedup vs baseline: 1.0144x; 1.0144x over previous
import jax
import jax.numpy as jnp
from jax import lax
from jax.experimental import pallas as pl
from jax.experimental.pallas import tpu as pltpu
from jax.experimental.pallas import tpu_sc as plsc

B = 16384
F = 26
FIELD = 40000
TOTAL_VOCAB = F * FIELD
NC = 2
NS = 16
L = 16
NW = NC * NS
BPW = B // NW
PER_W = F * BPW
NCHUNK = BPW // L
VSLICE = 65024
VLAST = TOTAL_VOCAB - (NS - 1) * VSLICE


def _sc_body(x_hbm, w_hbm, table_hbm, bias_hbm, out_hbm,
             idx_v, w_v, val_v, out_v, bias_v, spmem_t, sem, sem2, semw):
    c = lax.axis_index("c")
    s = lax.axis_index("s")
    wid = s * NC + c
    base = wid * BPW

    @pl.when(s < NS - 1)
    def _stage_main():
        pltpu.make_async_copy(
            table_hbm.at[0, pl.ds(s * VSLICE, VSLICE)],
            spmem_t.at[0, pl.ds(s * VSLICE, VSLICE)], sem2).start()

    @pl.when(s == NS - 1)
    def _stage_last():
        pltpu.make_async_copy(
            table_hbm.at[0, pl.ds((NS - 1) * VSLICE, VLAST)],
            spmem_t.at[0, pl.ds((NS - 1) * VSLICE, VLAST)], sem2).start()

    xcopies, wcopies = [], []
    for f in range(F):
        xcopies.append(pltpu.make_async_copy(
            x_hbm.at[f, pl.ds(base, BPW)], idx_v.at[pl.ds(f * BPW, BPW)], sem))
        wcopies.append(pltpu.make_async_copy(
            w_hbm.at[0, pl.ds(f * B + base, BPW)], w_v.at[pl.ds(f * BPW, BPW)],
            semw))
    for cp in xcopies:
        cp.start()
    for cp in wcopies:
        cp.start()
    pltpu.sync_copy(bias_hbm, bias_v)
    for cp in xcopies:
        cp.wait()

    @pl.when(s < NS - 1)
    def _wait_main():
        pltpu.make_async_copy(
            table_hbm.at[0, pl.ds(s * VSLICE, VSLICE)],
            spmem_t.at[0, pl.ds(s * VSLICE, VSLICE)], sem2).wait()

    @pl.when(s == NS - 1)
    def _wait_last():
        pltpu.make_async_copy(
            table_hbm.at[0, pl.ds((NS - 1) * VSLICE, VLAST)],
            spmem_t.at[0, pl.ds((NS - 1) * VSLICE, VLAST)], sem2).wait()

    FH = F // 2

    def _offsets(flo, fhi):
        def _addf(f, _):
            off = f * jnp.int32(FIELD)
            fb = f * BPW

            def _add(i, _, off=off, fb=fb):
                sl = pl.ds(fb + i * L, L)
                idx_v[sl] = idx_v[sl] + off
                return _

            return lax.fori_loop(0, NCHUNK, _add, _, unroll=4)

        lax.fori_loop(max(flo, 1), fhi, _addf, 0)

    _offsets(0, FH)

    plsc.subcore_barrier()
    g1 = pltpu.make_async_copy(
        spmem_t.at[0].at[idx_v.at[pl.ds(0, FH * BPW)]],
        val_v.at[pl.ds(0, FH * BPW)], sem)
    g1.start()
    _offsets(FH, F)
    g2 = pltpu.make_async_copy(
        spmem_t.at[0].at[idx_v.at[pl.ds(FH * BPW, (F - FH) * BPW)]],
        val_v.at[pl.ds(FH * BPW, (F - FH) * BPW)], sem)
    g2.start()
    for cp in wcopies:
        cp.wait()
    g1.wait()

    def _reduce_a(i, _):
        sl0 = pl.ds(i * L, L)
        acc0 = bias_v[...] + val_v[sl0] * w_v[sl0]
        sl1 = pl.ds(BPW + i * L, L)
        acc1 = val_v[sl1] * w_v[sl1]
        for f in range(2, FH):
            sl = pl.ds(f * BPW + i * L, L)
            if f % 2 == 0:
                acc0 = acc0 + val_v[sl] * w_v[sl]
            else:
                acc1 = acc1 + val_v[sl] * w_v[sl]
        out_v[sl0] = acc0 + acc1
        return _

    lax.fori_loop(0, NCHUNK, _reduce_a, 0)
    g2.wait()

    def _reduce_b(i, _):
        sl0 = pl.ds(i * L, L)
        acc0 = out_v[sl0]
        slh = pl.ds(FH * BPW + i * L, L)
        acc1 = val_v[slh] * w_v[slh]
        for f in range(FH + 1, F):
            sl = pl.ds(f * BPW + i * L, L)
            if f % 2 == 0:
                acc0 = acc0 + val_v[sl] * w_v[sl]
            else:
                acc1 = acc1 + val_v[sl] * w_v[sl]
        out_v[sl0] = acc0 + acc1
        return _

    lax.fori_loop(0, NCHUNK, _reduce_b, 0)

    pltpu.sync_copy(out_v, out_hbm.at[pl.ds(base, BPW)])


@jax.jit
def kernel(x, weight, fc_table, bias):
    x_t = x.astype(jnp.int32).T
    w_t = lax.reshape(weight, (1, B * F), dimensions=(2, 1, 0))
    table2 = fc_table.reshape(1, TOTAL_VOCAB)
    bias16 = jnp.broadcast_to(bias.reshape(1), (L,))

    mesh = plsc.VectorSubcoreMesh(core_axis_name="c", subcore_axis_name="s")
    out = pl.kernel(
        _sc_body,
        mesh=mesh,
        out_type=jax.ShapeDtypeStruct((B,), jnp.float32),
        scratch_types=[
            pltpu.VMEM((PER_W,), jnp.int32),
            pltpu.VMEM((PER_W,), jnp.float32),
            pltpu.VMEM((PER_W,), jnp.float32),
            pltpu.VMEM((BPW,), jnp.float32),
            pltpu.VMEM((L,), jnp.float32),
            pltpu.VMEM_SHARED((1, TOTAL_VOCAB), jnp.float32),
            pltpu.SemaphoreType.DMA,
            pltpu.SemaphoreType.DMA,
            pltpu.SemaphoreType.DMA,
        ],
    )(x_t, w_t, table2, bias16)
    return out.reshape(B, 1)
